# baseline (speedup 1.0000x reference)
import functools

import jax
import jax.numpy as jnp
from jax import lax
from jax.experimental import pallas as pl
from jax.experimental.pallas import tpu as pltpu
from jax.experimental.pallas import tpu_sc as plsc

NC = 2
NS = 16
NW = NC * NS
C = 128
NBUF = 3


def _log_1to2(x):
    s = (x - 1.0) / (x + 1.0)
    s2 = s * s
    p = 1.0 / 9.0
    p = p * s2 + 1.0 / 7.0
    p = p * s2 + 1.0 / 5.0
    p = p * s2 + 1.0 / 3.0
    p = p * s2 + 1.0
    return (2.0 * s) * p


def _sqrt(x):
    i = lax.bitcast_convert_type(x, jnp.int32)
    y = lax.bitcast_convert_type(jnp.int32(0x5F3759DF) - (i >> 1), jnp.float32)
    for _ in range(3):
        y = y * (1.5 - 0.5 * x * y * y)
    return x * y


def kernel(pos_idxs, ys, neg_idxs, num_neg, W):
    B = pos_idxs.shape[1]
    NT = neg_idxs.shape[1]
    V, D = W.shape
    TOT = B + NT
    ppw = TOT // NW
    nch = ppw // C
    assert ppw % C == 0 and nch % NBUF == 0 and D == 64 and V % 2 == 0
    scale = 1.0 / float(TOT)

    mesh = plsc.VectorSubcoreMesh(core_axis_name="c", subcore_axis_name="s")

    @functools.partial(
        pl.kernel,
        mesh=mesh,
        compiler_params=pltpu.CompilerParams(
            needs_layout_passes=False, use_tc_tiling_on_sc=True),
        out_type=jax.ShapeDtypeStruct((NW, 16), jnp.float32),
        scratch_types=(
            [pltpu.VMEM((C,), jnp.int32) for _ in range(4 * NBUF)]
            + [pltpu.VMEM((C,), jnp.float32) for _ in range(NBUF)]
            + [pltpu.VMEM((C, 2 * D), jnp.float32) for _ in range(2 * NBUF)]
            + [pltpu.VMEM((16,), jnp.float32)]
            + [pltpu.SemaphoreType.DMA for _ in range(2 * NBUF)]
        ),
    )
    def sc_loss(i0_h, i1_h, yse_h, w2_h, out_h, *refs):
        raw_u = refs[0:NBUF]
        raw_v = refs[NBUF:2 * NBUF]
        fet_u = refs[2 * NBUF:3 * NBUF]
        fet_v = refs[3 * NBUF:4 * NBUF]
        ysv = refs[4 * NBUF:5 * NBUF]
        urows = refs[5 * NBUF:6 * NBUF]
        vrows = refs[6 * NBUF:7 * NBUF]
        accv = refs[7 * NBUF]
        semu = refs[7 * NBUF + 1:7 * NBUF + 1 + NBUF]
        semv = refs[7 * NBUF + 1 + NBUF:7 * NBUF + 1 + 2 * NBUF]

        wid = lax.axis_index("s") * NC + lax.axis_index("c")
        tbase = wid * ppw
        lanes = lax.iota(jnp.int32, 16)

        def issue(c, r):
            base = tbase + c * C
            pltpu.sync_copy(i0_h.at[pl.ds(base, C)], raw_u[r])
            pltpu.sync_copy(i1_h.at[pl.ds(base, C)], raw_v[r])
            pltpu.sync_copy(yse_h.at[pl.ds(base, C)], ysv[r])
            for s in range(C // 16):
                sl = pl.ds(s * 16, 16)
                fet_u[r][sl] = raw_u[r][sl] >> 1
                fet_v[r][sl] = raw_v[r][sl] >> 1
            pltpu.async_copy(w2_h.at[fet_u[r]], urows[r], semu[r])
            pltpu.async_copy(w2_h.at[fet_v[r]], vrows[r], semv[r])

        def wait(r):
            pltpu.make_async_copy(w2_h.at[fet_u[r]], urows[r], semu[r]).wait()
            pltpu.make_async_copy(w2_h.at[fet_v[r]], vrows[r], semv[r]).wait()

        def compute(c, r, acc):
            def group(g, acc):
                rvec = g * 16 + lanes
                su = (plsc.load_gather(raw_u[r], [rvec]) & 1) * D
                sv = (plsc.load_gather(raw_v[r], [rvec]) & 1) * D
                dot = jnp.zeros((16,), jnp.float32)
                for j in range(D):
                    au = plsc.load_gather(urows[r], [rvec, su + j])
                    av = plsc.load_gather(vrows[r], [rvec, sv + j])
                    dot = dot + au * av
                pid = tbase + c * C + g * 16 + lanes
                z = jnp.where(pid < B, -dot, dot)
                t = jnp.exp(-jnp.abs(z))
                sp = jnp.maximum(z, 0.0) + _log_1to2(1.0 + t)
                yv = plsc.load_gather(ysv[r], [rvec])
                alpha = _log_1to2(1.0 + _sqrt(yv)) + 1.0
                return acc + alpha * sp

            return lax.fori_loop(0, C // 16, group, acc)

        for r in range(NBUF):
            issue(r, r)

        def step(k, acc):
            for r in range(NBUF):
                c = k * NBUF + r
                wait(r)
                acc = compute(c, r, acc)
                issue(c + NBUF, r)
            return acc

        acc = lax.fori_loop(0, nch // NBUF - 1, step,
                            jnp.zeros((16,), jnp.float32))
        for r in range(NBUF):
            c = nch - NBUF + r
            wait(r)
            acc = compute(c, r, acc)

        accv[...] = acc * scale
        pltpu.sync_copy(accv, out_h.at[wid])

    i0 = jnp.concatenate([pos_idxs[0], neg_idxs[0]])
    i1 = jnp.concatenate([pos_idxs[1], neg_idxs[1]])
    yse = jnp.concatenate([ys, jnp.zeros((NT,), jnp.float32)])
    w2 = W.reshape(V // 2, 2 * D)
    partials = sc_loss(i0, i1, yse, w2)
    return jnp.sum(partials)


# speedup vs baseline: 1.0354x; 1.0354x over previous
import functools

import jax
import jax.numpy as jnp
from jax import lax
from jax.experimental import pallas as pl
from jax.experimental.pallas import tpu as pltpu
from jax.experimental.pallas import tpu_sc as plsc

NC = 2
NS = 16
NW = NC * NS
C = 128
NBUF = 3


def _log_1to2(x):
    s = (x - 1.0) / (x + 1.0)
    s2 = s * s
    p = 1.0 / 9.0
    p = p * s2 + 1.0 / 7.0
    p = p * s2 + 1.0 / 5.0
    p = p * s2 + 1.0 / 3.0
    p = p * s2 + 1.0
    return (2.0 * s) * p


def _sqrt(x):
    i = lax.bitcast_convert_type(x, jnp.int32)
    y = lax.bitcast_convert_type(jnp.int32(0x5F3759DF) - (i >> 1), jnp.float32)
    for _ in range(3):
        y = y * (1.5 - 0.5 * x * y * y)
    return x * y


def kernel(pos_idxs, ys, neg_idxs, num_neg, W):
    B = pos_idxs.shape[1]
    NT = neg_idxs.shape[1]
    V, D = W.shape
    TOT = B + NT
    ppw = TOT // NW
    nch = ppw // C
    assert ppw % C == 0 and nch >= 2 * NBUF and D == 64 and V % 2 == 0
    scale = 1.0 / float(TOT)

    mesh = plsc.VectorSubcoreMesh(core_axis_name="c", subcore_axis_name="s")

    @functools.partial(
        pl.kernel,
        mesh=mesh,
        compiler_params=pltpu.CompilerParams(
            needs_layout_passes=False, use_tc_tiling_on_sc=True),
        out_type=jax.ShapeDtypeStruct((NW, 16), jnp.float32),
        scratch_types=(
            [pltpu.VMEM((ppw,), jnp.int32) for _ in range(4)]
            + [pltpu.VMEM((ppw,), jnp.float32)]
            + [pltpu.VMEM((C, 2 * D), jnp.float32) for _ in range(2 * NBUF)]
            + [pltpu.VMEM((16,), jnp.float32)]
            + [pltpu.SemaphoreType.DMA for _ in range(2 * NBUF)]
        ),
    )
    def sc_loss(i0_h, i1_h, yse_h, w2_h, out_h, *refs):
        rawu, rawv, fetu, fetv, ysa = refs[0:5]
        urows = refs[5:5 + NBUF]
        vrows = refs[5 + NBUF:5 + 2 * NBUF]
        accv = refs[5 + 2 * NBUF]
        semu = refs[6 + 2 * NBUF:6 + 2 * NBUF + NBUF]
        semv = refs[6 + 2 * NBUF + NBUF:6 + 2 * NBUF + 2 * NBUF]

        wid = lax.axis_index("s") * NC + lax.axis_index("c")
        tbase = wid * ppw
        lanes = lax.iota(jnp.int32, 16)

        pltpu.sync_copy(i0_h.at[pl.ds(tbase, ppw)], rawu)
        pltpu.sync_copy(i1_h.at[pl.ds(tbase, ppw)], rawv)
        pltpu.sync_copy(yse_h.at[pl.ds(tbase, ppw)], ysa)

        def shift_step(s, _):
            sl = pl.ds(s * 16, 16)
            fetu[sl] = rawu[sl] >> 1
            fetv[sl] = rawv[sl] >> 1
            return 0
        lax.fori_loop(0, ppw // 16, shift_step, 0)

        def fire(c, r):
            pltpu.async_copy(
                w2_h.at[fetu.at[pl.ds(c * C, C)]], urows[r], semu[r])
            pltpu.async_copy(
                w2_h.at[fetv.at[pl.ds(c * C, C)]], vrows[r], semv[r])

        def wait(c, r):
            pltpu.make_async_copy(
                w2_h.at[fetu.at[pl.ds(c * C, C)]], urows[r], semu[r]).wait()
            pltpu.make_async_copy(
                w2_h.at[fetv.at[pl.ds(c * C, C)]], vrows[r], semv[r]).wait()

        def compute(c, r, acc):
            def group(g, acc):
                rvec = g * 16 + lanes
                avec = c * C + rvec
                su = (plsc.load_gather(rawu, [avec]) & 1) * D
                sv = (plsc.load_gather(rawv, [avec]) & 1) * D
                dot = jnp.zeros((16,), jnp.float32)
                for j in range(D):
                    au = plsc.load_gather(urows[r], [rvec, su + j])
                    av = plsc.load_gather(vrows[r], [rvec, sv + j])
                    dot = dot + au * av
                z = jnp.where(tbase + avec < B, -dot, dot)
                t = jnp.exp(-jnp.abs(z))
                sp = jnp.maximum(z, 0.0) + _log_1to2(1.0 + t)
                yv = plsc.load_gather(ysa, [avec])
                alpha = _log_1to2(1.0 + _sqrt(yv)) + 1.0
                return acc + alpha * sp

            return lax.fori_loop(0, C // 16, group, acc)

        fire(0, 0)
        fire(1, 1)

        def step(k, acc):
            for r in range(NBUF):
                c = k * NBUF + r
                fire(c + 2, (r + 2) % NBUF)
                wait(c, r)
                acc = compute(c, r, acc)
            return acc

        assert nch % NBUF == 0
        acc = lax.fori_loop(0, nch // NBUF - 1, step,
                            jnp.zeros((16,), jnp.float32))
        c = nch - 3
        fire(nch - 1, (nch - 1) % NBUF)
        wait(c, c % NBUF)
        acc = compute(c, c % NBUF, acc)
        for c in range(nch - 2, nch):
            wait(c, c % NBUF)
            acc = compute(c, c % NBUF, acc)

        accv[...] = acc * scale
        pltpu.sync_copy(accv, out_h.at[wid])

    i0 = jnp.concatenate([pos_idxs[0], neg_idxs[0]])
    i1 = jnp.concatenate([pos_idxs[1], neg_idxs[1]])
    yse = jnp.concatenate([ys, jnp.zeros((NT,), jnp.float32)])
    w2 = W.reshape(V // 2, 2 * D)
    partials = sc_loss(i0, i1, yse, w2)
    return jnp.sum(partials)
